# baseline (speedup 1.0000x reference)
import functools

import jax
import jax.numpy as jnp
from jax import lax
from jax.experimental import pallas as pl
from jax.experimental.pallas import tpu as pltpu
from jax.experimental.pallas import tpu_sc as plsc

_B, _N, _F = 4, 50000, 128
_K = 25000
_NPAD = 51200
_CH = 3200
_VR = _CH // 16
_R = 2048
_SHIFTS = (0, 11, 22)
_T = 16
_NBLK = _K // 128
_TAIL = _K - _NBLK * 128

_TCBLK = 1600


def _score_body(x_ref, w_ref, o_ref):
    j = pl.program_id(1)
    x = x_ref[0]
    w = w_ref[...]
    s = jnp.dot(x, w, preferred_element_type=jnp.float32)
    u = lax.bitcast_convert_type(s, jnp.uint32)
    neg = u >= jnp.uint32(0x80000000)
    m = jnp.where(neg, ~u, u | jnp.uint32(0x80000000))
    dk = ~m
    rows = j * _TCBLK + lax.broadcasted_iota(jnp.int32, (_TCBLK, 1), 0)
    dk = jnp.where(rows < _N, dk, jnp.uint32(0xFFFFFFFF))
    o_ref[...] = dk[None]


def _scores_tc(inputs, score_proj):
    return pl.pallas_call(
        _score_body,
        grid=(_B, _NPAD // _TCBLK),
        in_specs=[
            pl.BlockSpec((1, _TCBLK, _F), lambda b, j: (b, j, 0)),
            pl.BlockSpec((_F, 1), lambda b, j: (0, 0)),
        ],
        out_specs=pl.BlockSpec((1, _TCBLK, 1), lambda b, j: (b, j, 0)),
        out_shape=jax.ShapeDtypeStruct((_B, _NPAD, 1), jnp.uint32),
    )(inputs, score_proj)


def _sc_body(dk_hbm, x_hbm, out_hbm,
             sA_k, sA_i, sB_k, sB_i, sHist,
             ck, ci, rnk, dstf, lh, offs, tot, cbr, haf,
             idxb, dkst, gate, rows):
    c = lax.axis_index("c")
    t = lax.axis_index("s")
    i16 = lax.iota(jnp.int32, 16)

    def radix_pass(b, sh, first, srck, srci, dstk, dsti):
        if first:
            pltpu.sync_copy(dk_hbm.at[pl.ds(b * _NPAD + t * _CH, _CH)], ck)

            def initci(v, carry):
                ci[pl.ds(v * 16, 16)] = t * _CH + v * 16 + i16
                return carry

            lax.fori_loop(0, _VR, initci, jnp.int32(0))
        else:
            pltpu.sync_copy(srck.at[pl.ds(t * _CH, _CH)], ck)
            pltpu.sync_copy(srci.at[pl.ds(t * _CH, _CH)], ci)

        def zh(i, carry):
            lh[pl.ds(i * 16, 16)] = jnp.zeros((16,), jnp.int32)
            return carry

        lax.fori_loop(0, _R // 16, zh, jnp.int32(0))

        def hb(v, carry):
            kv = ck[pl.ds(v * 16, 16)]
            d = ((kv >> sh) & jnp.uint32(_R - 1)).astype(jnp.int32)
            cnt, last = plsc.scan_count(d)
            prior = plsc.load_gather(lh, [d])
            rloc = prior + cnt - 1
            rnk[pl.ds(v * 16, 16)] = rloc
            plsc.store_scatter(lh, [d], rloc + 1, mask=last)
            return carry

        lax.fori_loop(0, _VR, hb, jnp.int32(0))

        pltpu.sync_copy(lh, sHist.at[pl.ds(t * _R, _R)])
        plsc.subcore_barrier()
        pltpu.sync_copy(sHist, haf)

        def ob(dv, carry):
            s = jnp.zeros((16,), jnp.int32)
            cb = jnp.zeros((16,), jnp.int32)
            for tt in range(_T):
                h = haf[pl.ds(tt * _R + dv * 16, 16)]
                s = s + h
                cb = cb + h * (jnp.int32(tt) < t).astype(jnp.int32)
            tot[pl.ds(dv * 16, 16)] = s
            cbr[pl.ds(dv * 16, 16)] = cb
            return carry

        lax.fori_loop(0, _R // 16, ob, jnp.int32(0))

        def pb(dv, carry):
            tv = tot[pl.ds(dv * 16, 16)]
            cs = plsc.cumsum(tv)
            excl = cs - tv + carry
            offs[pl.ds(dv * 16, 16)] = excl + cbr[pl.ds(dv * 16, 16)]
            return carry + jnp.sum(tv)

        lax.fori_loop(0, _R // 16, pb, jnp.int32(0))

        def db(v, carry):
            kv = ck[pl.ds(v * 16, 16)]
            d = ((kv >> sh) & jnp.uint32(_R - 1)).astype(jnp.int32)
            off = plsc.load_gather(offs, [d])
            dstf[pl.ds(v * 16, 16)] = off + rnk[pl.ds(v * 16, 16)]
            return carry

        lax.fori_loop(0, _VR, db, jnp.int32(0))

        pltpu.sync_copy(ck, dstk.at[dstf])
        pltpu.sync_copy(ci, dsti.at[dstf])
        plsc.subcore_barrier()

    def topk_block(b, p0, n_load, n_write):
        if n_load == 128:
            idx_sl, dk_sl, rows_sl = idxb, dkst, rows
        else:
            idx_sl = idxb.at[pl.ds(0, n_load)]
            dk_sl = dkst.at[pl.ds(0, n_load)]
            rows_sl = rows.at[pl.ds(0, n_load)]
        pltpu.sync_copy(sA_i.at[pl.ds(p0, n_load)], idx_sl)
        pltpu.sync_copy(sA_k.at[pl.ds(p0, n_load)], dk_sl)

        def adj(u, carry):
            idxb[pl.ds(u * 16, 16)] = idxb[pl.ds(u * 16, 16)] + b * _N
            return carry

        lax.fori_loop(0, n_load // 16, adj, jnp.int32(0))

        pltpu.sync_copy(x_hbm.at[idx_sl], rows_sl)

        def gb(u, carry):
            m = ~dkst[pl.ds(u * 16, 16)]
            pos = m >= jnp.uint32(0x80000000)
            uu = jnp.where(pos, m & jnp.uint32(0x7FFFFFFF), ~m)
            s = lax.bitcast_convert_type(uu, jnp.float32)
            gate[pl.ds(u * 16, 16)] = 1.0 / (1.0 + jnp.exp(-s))
            return carry

        lax.fori_loop(0, n_load // 16, gb, jnp.int32(0))

        def rb(i, carry):
            g = plsc.load_gather(gate, [jnp.full((16,), i, jnp.int32)])
            for u8 in range(_F // 16):
                rows[i, pl.ds(u8 * 16, 16)] = rows[i, pl.ds(u8 * 16, 16)] * g
            return carry

        lax.fori_loop(0, n_write, rb, jnp.int32(0))

        out_sl = rows if n_write == 128 else rows.at[pl.ds(0, n_write)]
        pltpu.sync_copy(out_sl, out_hbm.at[pl.ds(b * _K + p0, n_write)])

    for bl in range(2):
        b = c * 2 + bl
        radix_pass(b, _SHIFTS[0], True, None, None, sA_k, sA_i)
        radix_pass(b, _SHIFTS[1], False, sA_k, sA_i, sB_k, sB_i)
        radix_pass(b, _SHIFTS[2], False, sB_k, sB_i, sA_k, sA_i)

        for jj in range(13):
            g = t + jj * _T
            pl.when(g < _NBLK)(
                functools.partial(topk_block, b, g * 128, 128, 128))
        pl.when(t == 3)(
            functools.partial(topk_block, b, _NBLK * 128, 48, _TAIL))
        plsc.subcore_barrier()


_sc_sort_gather = pl.kernel(
    _sc_body,
    out_type=jax.ShapeDtypeStruct((_B * _K, _F), jnp.float32),
    mesh=plsc.VectorSubcoreMesh(core_axis_name="c", subcore_axis_name="s"),
    compiler_params=pltpu.CompilerParams(needs_layout_passes=False),
    scratch_types=[
        pltpu.VMEM_SHARED((_NPAD,), jnp.uint32),
        pltpu.VMEM_SHARED((_NPAD,), jnp.int32),
        pltpu.VMEM_SHARED((_NPAD,), jnp.uint32),
        pltpu.VMEM_SHARED((_NPAD,), jnp.int32),
        pltpu.VMEM_SHARED((_T * _R,), jnp.int32),
        pltpu.VMEM((_CH,), jnp.uint32),
        pltpu.VMEM((_CH,), jnp.int32),
        pltpu.VMEM((_CH,), jnp.int32),
        pltpu.VMEM((_CH,), jnp.int32),
        pltpu.VMEM((_R,), jnp.int32),
        pltpu.VMEM((_R,), jnp.int32),
        pltpu.VMEM((_R,), jnp.int32),
        pltpu.VMEM((_R,), jnp.int32),
        pltpu.VMEM((_T * _R,), jnp.int32),
        pltpu.VMEM((128,), jnp.int32),
        pltpu.VMEM((128,), jnp.uint32),
        pltpu.VMEM((128,), jnp.float32),
        pltpu.VMEM((128, _F), jnp.float32),
    ],
)


def kernel(inputs, score_proj):
    dk = _scores_tc(inputs, score_proj).reshape(_B * _NPAD)
    xflat = inputs.reshape(_B * _N, _F)
    out = _sc_sort_gather(dk, xflat)
    return out.reshape(_B, _K, _F)


# speedup vs baseline: 1.5154x; 1.5154x over previous
import functools

import jax
import jax.numpy as jnp
from jax import lax
from jax.experimental import pallas as pl
from jax.experimental.pallas import tpu as pltpu
from jax.experimental.pallas import tpu_sc as plsc

_B, _N, _F = 4, 50000, 128
_K = 25000
_NPAD = 51200
_CH = 3200
_VR = _CH // 16
_R = 2048
_SHIFTS = (0, 11, 22)
_T = 16
_NBLK = _K // 128
_TAIL = _K - _NBLK * 128

_TCBLK = 10240


def _score_body(x_ref, w_ref, o_ref):
    j = pl.program_id(1)
    x = x_ref[0]
    w = w_ref[...]
    s = jnp.dot(x, w, preferred_element_type=jnp.float32)
    st = s.reshape(_TCBLK // 128, 128)
    u = lax.bitcast_convert_type(st, jnp.uint32)
    neg = u >= jnp.uint32(0x80000000)
    m = jnp.where(neg, ~u, u | jnp.uint32(0x80000000))
    dk = ~m
    rows = (j * _TCBLK
            + lax.broadcasted_iota(jnp.int32, (_TCBLK // 128, 128), 0) * 128
            + lax.broadcasted_iota(jnp.int32, (_TCBLK // 128, 128), 1))
    dk = jnp.where(rows < _N, dk, jnp.uint32(0xFFFFFFFF))
    o_ref[...] = dk[None]


def _scores_tc(inputs, score_proj):
    return pl.pallas_call(
        _score_body,
        grid=(_B, _NPAD // _TCBLK),
        in_specs=[
            pl.BlockSpec((1, _TCBLK, _F), lambda b, j: (b, j, 0)),
            pl.BlockSpec((_F, 1), lambda b, j: (0, 0)),
        ],
        out_specs=pl.BlockSpec((1, _TCBLK // 128, 128), lambda b, j: (b, j, 0)),
        out_shape=jax.ShapeDtypeStruct((_B, _NPAD // 128, 128), jnp.uint32),
    )(inputs, score_proj)


def _sc_body(dk_hbm, x_hbm, out_hbm,
             sA_k, sA_i, sB_k, sB_i, sHist,
             ck, ci, rnk, dstf, lh, offs, tot, cbr, haf,
             idxb, dkst, gate, rows):
    c = lax.axis_index("c")
    t = lax.axis_index("s")
    i16 = lax.iota(jnp.int32, 16)

    def radix_pass(b, sh, first, srck, srci, dstk, dsti):
        if first:
            pltpu.sync_copy(dk_hbm.at[pl.ds(b * _NPAD + t * _CH, _CH)], ck)

            def initci(v, carry):
                ci[pl.ds(v * 16, 16)] = t * _CH + v * 16 + i16
                return carry

            lax.fori_loop(0, _VR, initci, jnp.int32(0))
        else:
            pltpu.sync_copy(srck.at[pl.ds(t * _CH, _CH)], ck)
            pltpu.sync_copy(srci.at[pl.ds(t * _CH, _CH)], ci)

        def zh(i, carry):
            lh[pl.ds(i * 16, 16)] = jnp.zeros((16,), jnp.int32)
            return carry

        lax.fori_loop(0, _R // 16, zh, jnp.int32(0))

        def hb(v, carry):
            kv = ck[pl.ds(v * 16, 16)]
            d = ((kv >> sh) & jnp.uint32(_R - 1)).astype(jnp.int32)
            cnt, last = plsc.scan_count(d)
            prior = plsc.load_gather(lh, [d])
            rloc = prior + cnt - 1
            rnk[pl.ds(v * 16, 16)] = rloc
            plsc.store_scatter(lh, [d], rloc + 1, mask=last)
            return carry

        lax.fori_loop(0, _VR, hb, jnp.int32(0))

        pltpu.sync_copy(lh, sHist.at[pl.ds(t * _R, _R)])
        plsc.subcore_barrier()
        pltpu.sync_copy(sHist, haf)

        def ob(dv, carry):
            s = jnp.zeros((16,), jnp.int32)
            cb = jnp.zeros((16,), jnp.int32)
            for tt in range(_T):
                h = haf[pl.ds(tt * _R + dv * 16, 16)]
                s = s + h
                cb = cb + h * (jnp.int32(tt) < t).astype(jnp.int32)
            tot[pl.ds(dv * 16, 16)] = s
            cbr[pl.ds(dv * 16, 16)] = cb
            return carry

        lax.fori_loop(0, _R // 16, ob, jnp.int32(0))

        def pb(dv, carry):
            tv = tot[pl.ds(dv * 16, 16)]
            cs = plsc.cumsum(tv)
            excl = cs - tv + carry
            offs[pl.ds(dv * 16, 16)] = excl + cbr[pl.ds(dv * 16, 16)]
            return carry + jnp.sum(tv)

        lax.fori_loop(0, _R // 16, pb, jnp.int32(0))

        def db(v, carry):
            kv = ck[pl.ds(v * 16, 16)]
            d = ((kv >> sh) & jnp.uint32(_R - 1)).astype(jnp.int32)
            off = plsc.load_gather(offs, [d])
            dstf[pl.ds(v * 16, 16)] = off + rnk[pl.ds(v * 16, 16)]
            return carry

        lax.fori_loop(0, _VR, db, jnp.int32(0))

        pltpu.sync_copy(ck, dstk.at[dstf])
        pltpu.sync_copy(ci, dsti.at[dstf])
        plsc.subcore_barrier()

    def topk_block(b, p0, n_load, n_write):
        if n_load == 128:
            idx_sl, dk_sl, rows_sl = idxb, dkst, rows
        else:
            idx_sl = idxb.at[pl.ds(0, n_load)]
            dk_sl = dkst.at[pl.ds(0, n_load)]
            rows_sl = rows.at[pl.ds(0, n_load)]
        pltpu.sync_copy(sA_i.at[pl.ds(p0, n_load)], idx_sl)
        pltpu.sync_copy(sA_k.at[pl.ds(p0, n_load)], dk_sl)

        def adj(u, carry):
            idxb[pl.ds(u * 16, 16)] = idxb[pl.ds(u * 16, 16)] + b * _N
            return carry

        lax.fori_loop(0, n_load // 16, adj, jnp.int32(0))

        pltpu.sync_copy(x_hbm.at[idx_sl], rows_sl)

        def gb(u, carry):
            m = ~dkst[pl.ds(u * 16, 16)]
            pos = m >= jnp.uint32(0x80000000)
            uu = jnp.where(pos, m & jnp.uint32(0x7FFFFFFF), ~m)
            s = lax.bitcast_convert_type(uu, jnp.float32)
            gate[pl.ds(u * 16, 16)] = 1.0 / (1.0 + jnp.exp(-s))
            return carry

        lax.fori_loop(0, n_load // 16, gb, jnp.int32(0))

        def rb(i, carry):
            g = plsc.load_gather(gate, [jnp.full((16,), i, jnp.int32)])
            for u8 in range(_F // 16):
                rows[i, pl.ds(u8 * 16, 16)] = rows[i, pl.ds(u8 * 16, 16)] * g
            return carry

        lax.fori_loop(0, n_write, rb, jnp.int32(0))

        out_sl = rows if n_write == 128 else rows.at[pl.ds(0, n_write)]
        pltpu.sync_copy(out_sl, out_hbm.at[pl.ds(b * _K + p0, n_write)])

    for bl in range(2):
        b = c * 2 + bl
        radix_pass(b, _SHIFTS[0], True, None, None, sA_k, sA_i)
        radix_pass(b, _SHIFTS[1], False, sA_k, sA_i, sB_k, sB_i)
        radix_pass(b, _SHIFTS[2], False, sB_k, sB_i, sA_k, sA_i)

        for jj in range(13):
            g = t + jj * _T
            pl.when(g < _NBLK)(
                functools.partial(topk_block, b, g * 128, 128, 128))
        pl.when(t == 3)(
            functools.partial(topk_block, b, _NBLK * 128, 48, _TAIL))
        plsc.subcore_barrier()


_sc_sort_gather = pl.kernel(
    _sc_body,
    out_type=jax.ShapeDtypeStruct((_B * _K, _F), jnp.float32),
    mesh=plsc.VectorSubcoreMesh(core_axis_name="c", subcore_axis_name="s"),
    compiler_params=pltpu.CompilerParams(needs_layout_passes=False),
    scratch_types=[
        pltpu.VMEM_SHARED((_NPAD,), jnp.uint32),
        pltpu.VMEM_SHARED((_NPAD,), jnp.int32),
        pltpu.VMEM_SHARED((_NPAD,), jnp.uint32),
        pltpu.VMEM_SHARED((_NPAD,), jnp.int32),
        pltpu.VMEM_SHARED((_T * _R,), jnp.int32),
        pltpu.VMEM((_CH,), jnp.uint32),
        pltpu.VMEM((_CH,), jnp.int32),
        pltpu.VMEM((_CH,), jnp.int32),
        pltpu.VMEM((_CH,), jnp.int32),
        pltpu.VMEM((_R,), jnp.int32),
        pltpu.VMEM((_R,), jnp.int32),
        pltpu.VMEM((_R,), jnp.int32),
        pltpu.VMEM((_R,), jnp.int32),
        pltpu.VMEM((_T * _R,), jnp.int32),
        pltpu.VMEM((128,), jnp.int32),
        pltpu.VMEM((128,), jnp.uint32),
        pltpu.VMEM((128,), jnp.float32),
        pltpu.VMEM((128, _F), jnp.float32),
    ],
)


def kernel(inputs, score_proj):
    dk = _scores_tc(inputs, score_proj).reshape(_B * _NPAD)
    xflat = inputs.reshape(_B * _N, _F)
    out = _sc_sort_gather(dk, xflat)
    return out.reshape(_B, _K, _F)


# speedup vs baseline: 1.8086x; 1.1935x over previous
import functools

import jax
import jax.numpy as jnp
from jax import lax
from jax.experimental import pallas as pl
from jax.experimental.pallas import tpu as pltpu
from jax.experimental.pallas import tpu_sc as plsc

_B, _N, _F = 4, 50000, 128
_K = 25000
_NPAD = 51200
_CH = 3200
_VR = _CH // 16
_R = 2048
_SHIFTS = (0, 11, 22)
_T = 16
_NBLK = _K // 128
_TAIL = _K - _NBLK * 128

_TCBLK = 25600


def _score_body(x_ref, w_ref, o_ref):
    j = pl.program_id(1)
    x = x_ref[0]
    w = w_ref[...]
    s = jnp.dot(x, w, preferred_element_type=jnp.float32)
    st = s.reshape(_TCBLK // 128, 128)
    u = lax.bitcast_convert_type(st, jnp.uint32)
    neg = u >= jnp.uint32(0x80000000)
    m = jnp.where(neg, ~u, u | jnp.uint32(0x80000000))
    dk = ~m
    rows = (j * _TCBLK
            + lax.broadcasted_iota(jnp.int32, (_TCBLK // 128, 128), 0) * 128
            + lax.broadcasted_iota(jnp.int32, (_TCBLK // 128, 128), 1))
    dk = jnp.where(rows < _N, dk, jnp.uint32(0xFFFFFFFF))
    o_ref[...] = dk[None]


def _scores_tc(inputs, score_proj):
    return pl.pallas_call(
        _score_body,
        grid=(_B, _NPAD // _TCBLK),
        in_specs=[
            pl.BlockSpec((1, _TCBLK, _F), lambda b, j: (b, j, 0)),
            pl.BlockSpec((_F, 1), lambda b, j: (0, 0)),
        ],
        out_specs=pl.BlockSpec((1, _TCBLK // 128, 128), lambda b, j: (b, j, 0)),
        out_shape=jax.ShapeDtypeStruct((_B, _NPAD // 128, 128), jnp.uint32),
    )(inputs, score_proj)


def _sc_body(dk_hbm, x_hbm, out_hbm,
             sA_k, sA_i, sB_k, sB_i, sHist,
             ck, ci, rnk, dstf, lh, offs, tot, cbr, haf,
             idxb0, idxb1, dkst0, dkst1, gate0, gate1, rows0, rows1,
             gsem0, gsem1, wsem0, wsem1):
    c = lax.axis_index("c")
    t = lax.axis_index("s")
    i16 = lax.iota(jnp.int32, 16)
    idxb = (idxb0, idxb1)
    dkst = (dkst0, dkst1)
    gate = (gate0, gate1)
    rows = (rows0, rows1)
    gsem = (gsem0, gsem1)
    wsem = (wsem0, wsem1)

    def radix_pass(b, sh, first, srck, srci, dstk, dsti):
        if first:
            pltpu.sync_copy(dk_hbm.at[pl.ds(b * _NPAD + t * _CH, _CH)], ck)

            def initci(v, carry):
                ci[pl.ds(v * 16, 16)] = t * _CH + v * 16 + i16
                return carry

            lax.fori_loop(0, _VR, initci, jnp.int32(0))
        else:
            pltpu.sync_copy(srck.at[pl.ds(t * _CH, _CH)], ck)
            pltpu.sync_copy(srci.at[pl.ds(t * _CH, _CH)], ci)

        def zh(i, carry):
            lh[pl.ds(i * 16, 16)] = jnp.zeros((16,), jnp.int32)
            return carry

        lax.fori_loop(0, _R // 16, zh, jnp.int32(0))

        def hb(v, carry):
            kv = ck[pl.ds(v * 16, 16)]
            d = ((kv >> sh) & jnp.uint32(_R - 1)).astype(jnp.int32)
            cnt, last = plsc.scan_count(d)
            prior = plsc.load_gather(lh, [d])
            rloc = prior + cnt - 1
            rnk[pl.ds(v * 16, 16)] = rloc
            plsc.store_scatter(lh, [d], rloc + 1, mask=last)
            return carry

        lax.fori_loop(0, _VR, hb, jnp.int32(0))

        pltpu.sync_copy(lh, sHist.at[pl.ds(t * _R, _R)])
        plsc.subcore_barrier()
        pltpu.sync_copy(sHist, haf)

        def ob(dv, carry):
            s = jnp.zeros((16,), jnp.int32)
            cb = jnp.zeros((16,), jnp.int32)
            for tt in range(_T):
                h = haf[pl.ds(tt * _R + dv * 16, 16)]
                s = s + h
                cb = cb + h * (jnp.int32(tt) < t).astype(jnp.int32)
            tot[pl.ds(dv * 16, 16)] = s
            cbr[pl.ds(dv * 16, 16)] = cb
            return carry

        lax.fori_loop(0, _R // 16, ob, jnp.int32(0))

        def pb(dv, carry):
            tv = tot[pl.ds(dv * 16, 16)]
            cs = plsc.cumsum(tv)
            excl = cs - tv + carry
            offs[pl.ds(dv * 16, 16)] = excl + cbr[pl.ds(dv * 16, 16)]
            return carry + jnp.sum(tv)

        lax.fori_loop(0, _R // 16, pb, jnp.int32(0))

        def db(v, carry):
            kv = ck[pl.ds(v * 16, 16)]
            d = ((kv >> sh) & jnp.uint32(_R - 1)).astype(jnp.int32)
            off = plsc.load_gather(offs, [d])
            dstf[pl.ds(v * 16, 16)] = off + rnk[pl.ds(v * 16, 16)]
            return carry

        lax.fori_loop(0, _VR, db, jnp.int32(0))

        pltpu.sync_copy(ck, dstk.at[dstf])
        pltpu.sync_copy(ci, dsti.at[dstf])
        plsc.subcore_barrier()

    def stage(idx_b, dk_b, gate_b, b, p0, n_load):
        if n_load == 128:
            idx_sl, dk_sl = idx_b, dk_b
        else:
            idx_sl = idx_b.at[pl.ds(0, n_load)]
            dk_sl = dk_b.at[pl.ds(0, n_load)]
        pltpu.sync_copy(sA_i.at[pl.ds(p0, n_load)], idx_sl)
        pltpu.sync_copy(sA_k.at[pl.ds(p0, n_load)], dk_sl)

        def adj(u, carry):
            idx_b[pl.ds(u * 16, 16)] = idx_b[pl.ds(u * 16, 16)] + b * _N
            return carry

        lax.fori_loop(0, n_load // 16, adj, jnp.int32(0))

        def gb(u, carry):
            m = ~dk_b[pl.ds(u * 16, 16)]
            pos = m >= jnp.uint32(0x80000000)
            uu = jnp.where(pos, m & jnp.uint32(0x7FFFFFFF), ~m)
            s = lax.bitcast_convert_type(uu, jnp.float32)
            gate_b[pl.ds(u * 16, 16)] = 1.0 / (1.0 + jnp.exp(-s))
            return carry

        lax.fori_loop(0, n_load // 16, gb, jnp.int32(0))
        return idx_sl

    def mult(rows_b, gate_b, n_write):
        def rb(i, carry):
            g = plsc.load_gather(gate_b, [jnp.full((16,), i, jnp.int32)])
            for u8 in range(_F // 16):
                rows_b[i, pl.ds(u8 * 16, 16)] = (
                    rows_b[i, pl.ds(u8 * 16, 16)] * g)
            return carry

        lax.fori_loop(0, n_write, rb, jnp.int32(0))

    def topk_block(b, p0, n_load, n_write):
        idx_sl = stage(idxb[0], dkst[0], gate[0], b, p0, n_load)
        rows_sl = rows[0] if n_load == 128 else rows[0].at[pl.ds(0, n_load)]
        pltpu.sync_copy(x_hbm.at[idx_sl], rows_sl)
        mult(rows[0], gate[0], n_write)
        out_sl = rows[0] if n_write == 128 else rows[0].at[pl.ds(0, n_write)]
        pltpu.sync_copy(out_sl, out_hbm.at[pl.ds(b * _K + p0, n_write)])

    for bl in range(2):
        b = c * 2 + bl
        radix_pass(b, _SHIFTS[0], True, None, None, sA_k, sA_i)
        radix_pass(b, _SHIFTS[1], False, sA_k, sA_i, sB_k, sB_i)
        radix_pass(b, _SHIFTS[2], False, sB_k, sB_i, sA_k, sA_i)

        gd = [None, None]
        wd = [None, None]
        stage(idxb[0], dkst[0], gate[0], b, t * 128, 128)
        gd[0] = pltpu.async_copy(x_hbm.at[idxb[0]], rows[0], gsem[0])
        for jj in range(1, 12):
            buf, pbuf = jj % 2, (jj - 1) % 2
            if jj >= 2:
                wd[buf].wait()
            stage(idxb[buf], dkst[buf], gate[buf], b, (t + jj * _T) * 128, 128)
            gd[buf] = pltpu.async_copy(x_hbm.at[idxb[buf]], rows[buf],
                                       gsem[buf])
            gd[pbuf].wait()
            mult(rows[pbuf], gate[pbuf], 128)
            wd[pbuf] = pltpu.async_copy(
                rows[pbuf],
                out_hbm.at[pl.ds(b * _K + (t + (jj - 1) * _T) * 128, 128)],
                wsem[pbuf])
        gd[1].wait()
        mult(rows[1], gate[1], 128)
        wd[1] = pltpu.async_copy(
            rows[1], out_hbm.at[pl.ds(b * _K + (t + 11 * _T) * 128, 128)],
            wsem[1])
        wd[0].wait()
        wd[1].wait()
        pl.when(t < _NBLK - 12 * _T)(
            functools.partial(topk_block, b, (t + 12 * _T) * 128, 128, 128))
        pl.when(t == 3)(
            functools.partial(topk_block, b, _NBLK * 128, 48, _TAIL))
        plsc.subcore_barrier()


_sc_sort_gather = pl.kernel(
    _sc_body,
    out_type=jax.ShapeDtypeStruct((_B * _K, _F), jnp.float32),
    mesh=plsc.VectorSubcoreMesh(core_axis_name="c", subcore_axis_name="s"),
    compiler_params=pltpu.CompilerParams(needs_layout_passes=False),
    scratch_types=[
        pltpu.VMEM_SHARED((_NPAD,), jnp.uint32),
        pltpu.VMEM_SHARED((_NPAD,), jnp.int32),
        pltpu.VMEM_SHARED((_NPAD,), jnp.uint32),
        pltpu.VMEM_SHARED((_NPAD,), jnp.int32),
        pltpu.VMEM_SHARED((_T * _R,), jnp.int32),
        pltpu.VMEM((_CH,), jnp.uint32),
        pltpu.VMEM((_CH,), jnp.int32),
        pltpu.VMEM((_CH,), jnp.int32),
        pltpu.VMEM((_CH,), jnp.int32),
        pltpu.VMEM((_R,), jnp.int32),
        pltpu.VMEM((_R,), jnp.int32),
        pltpu.VMEM((_R,), jnp.int32),
        pltpu.VMEM((_R,), jnp.int32),
        pltpu.VMEM((_T * _R,), jnp.int32),
        pltpu.VMEM((128,), jnp.int32),
        pltpu.VMEM((128,), jnp.int32),
        pltpu.VMEM((128,), jnp.uint32),
        pltpu.VMEM((128,), jnp.uint32),
        pltpu.VMEM((128,), jnp.float32),
        pltpu.VMEM((128,), jnp.float32),
        pltpu.VMEM((128, _F), jnp.float32),
        pltpu.VMEM((128, _F), jnp.float32),
        pltpu.SemaphoreType.DMA,
        pltpu.SemaphoreType.DMA,
        pltpu.SemaphoreType.DMA,
        pltpu.SemaphoreType.DMA,
    ],
)


def kernel(inputs, score_proj):
    dk = _scores_tc(inputs, score_proj).reshape(_B * _NPAD)
    xflat = inputs.reshape(_B * _N, _F)
    out = _sc_sort_gather(dk, xflat)
    return out.reshape(_B, _K, _F)


# speedup vs baseline: 1.8631x; 1.0301x over previous
import functools

import jax
import jax.numpy as jnp
from jax import lax
from jax.experimental import pallas as pl
from jax.experimental.pallas import tpu as pltpu
from jax.experimental.pallas import tpu_sc as plsc

_B, _N, _F = 4, 50000, 128
_K = 25000
_NPAD = 51200
_CH = 3200
_VR = _CH // 16
_R = 256
_SHIFTS = (0, 8, 16, 24)
_T = 16
_NBLK = _K // 128
_TAIL = _K - _NBLK * 128

_TCBLK = 25600


def _score_body(x_ref, w_ref, o_ref):
    j = pl.program_id(1)
    x = x_ref[0]
    w = w_ref[...]
    s = jnp.dot(x, w, preferred_element_type=jnp.float32)
    st = s.reshape(_TCBLK // 128, 128)
    u = lax.bitcast_convert_type(st, jnp.uint32)
    neg = u >= jnp.uint32(0x80000000)
    m = jnp.where(neg, ~u, u | jnp.uint32(0x80000000))
    dk = ~m
    rows = (j * _TCBLK
            + lax.broadcasted_iota(jnp.int32, (_TCBLK // 128, 128), 0) * 128
            + lax.broadcasted_iota(jnp.int32, (_TCBLK // 128, 128), 1))
    dk = jnp.where(rows < _N, dk, jnp.uint32(0xFFFFFFFF))
    o_ref[...] = dk[None]


def _scores_tc(inputs, score_proj):
    return pl.pallas_call(
        _score_body,
        grid=(_B, _NPAD // _TCBLK),
        in_specs=[
            pl.BlockSpec((1, _TCBLK, _F), lambda b, j: (b, j, 0)),
            pl.BlockSpec((_F, 1), lambda b, j: (0, 0)),
        ],
        out_specs=pl.BlockSpec((1, _TCBLK // 128, 128), lambda b, j: (b, j, 0)),
        out_shape=jax.ShapeDtypeStruct((_B, _NPAD // 128, 128), jnp.uint32),
    )(inputs, score_proj)


def _sc_body(dk_hbm, x_hbm, out_hbm,
             sA_k, sA_i, sB_k, sB_i, sHist,
             ck, ci, rnk, dstf, lh, offs, tot, cbr, haf,
             idxb0, idxb1, dkst0, dkst1, gate0, gate1, rows0, rows1,
             gsem0, gsem1, wsem0, wsem1):
    c = lax.axis_index("c")
    t = lax.axis_index("s")
    i16 = lax.iota(jnp.int32, 16)
    idxb = (idxb0, idxb1)
    dkst = (dkst0, dkst1)
    gate = (gate0, gate1)
    rows = (rows0, rows1)
    gsem = (gsem0, gsem1)
    wsem = (wsem0, wsem1)

    def radix_pass(b, sh, first, srck, srci, dstk, dsti):
        if first:
            pltpu.sync_copy(dk_hbm.at[pl.ds(b * _NPAD + t * _CH, _CH)], ck)

            def initci(v, carry):
                ci[pl.ds(v * 16, 16)] = t * _CH + v * 16 + i16
                return carry

            lax.fori_loop(0, _VR, initci, jnp.int32(0))
        else:
            pltpu.sync_copy(srck.at[pl.ds(t * _CH, _CH)], ck)
            pltpu.sync_copy(srci.at[pl.ds(t * _CH, _CH)], ci)

        def zh(i, carry):
            lh[pl.ds(i * 16, 16)] = jnp.zeros((16,), jnp.int32)
            return carry

        lax.fori_loop(0, _R // 16, zh, jnp.int32(0))

        def hb(v, carry):
            kv = ck[pl.ds(v * 16, 16)]
            d = ((kv >> sh) & jnp.uint32(_R - 1)).astype(jnp.int32)
            cnt, last = plsc.scan_count(d)
            prior = plsc.load_gather(lh, [d])
            rloc = prior + cnt - 1
            rnk[pl.ds(v * 16, 16)] = rloc
            plsc.store_scatter(lh, [d], rloc + 1, mask=last)
            return carry

        lax.fori_loop(0, _VR, hb, jnp.int32(0))

        pltpu.sync_copy(lh, sHist.at[pl.ds(t * _R, _R)])
        plsc.subcore_barrier()
        pltpu.sync_copy(sHist, haf)

        def ob(dv, carry):
            s = jnp.zeros((16,), jnp.int32)
            cb = jnp.zeros((16,), jnp.int32)
            for tt in range(_T):
                h = haf[pl.ds(tt * _R + dv * 16, 16)]
                s = s + h
                cb = cb + h * (jnp.int32(tt) < t).astype(jnp.int32)
            tot[pl.ds(dv * 16, 16)] = s
            cbr[pl.ds(dv * 16, 16)] = cb
            return carry

        lax.fori_loop(0, _R // 16, ob, jnp.int32(0))

        def pb(dv, carry):
            tv = tot[pl.ds(dv * 16, 16)]
            cs = plsc.cumsum(tv)
            excl = cs - tv + carry
            offs[pl.ds(dv * 16, 16)] = excl + cbr[pl.ds(dv * 16, 16)]
            return carry + jnp.sum(tv)

        lax.fori_loop(0, _R // 16, pb, jnp.int32(0))

        def db(v, carry):
            kv = ck[pl.ds(v * 16, 16)]
            d = ((kv >> sh) & jnp.uint32(_R - 1)).astype(jnp.int32)
            off = plsc.load_gather(offs, [d])
            dstf[pl.ds(v * 16, 16)] = off + rnk[pl.ds(v * 16, 16)]
            return carry

        lax.fori_loop(0, _VR, db, jnp.int32(0))

        pltpu.sync_copy(ck, dstk.at[dstf])
        pltpu.sync_copy(ci, dsti.at[dstf])
        plsc.subcore_barrier()

    def stage(idx_b, dk_b, gate_b, b, p0, n_load):
        if n_load == 128:
            idx_sl, dk_sl = idx_b, dk_b
        else:
            idx_sl = idx_b.at[pl.ds(0, n_load)]
            dk_sl = dk_b.at[pl.ds(0, n_load)]
        pltpu.sync_copy(sA_i.at[pl.ds(p0, n_load)], idx_sl)
        pltpu.sync_copy(sA_k.at[pl.ds(p0, n_load)], dk_sl)

        def adj(u, carry):
            idx_b[pl.ds(u * 16, 16)] = idx_b[pl.ds(u * 16, 16)] + b * _N
            return carry

        lax.fori_loop(0, n_load // 16, adj, jnp.int32(0))

        def gb(u, carry):
            m = ~dk_b[pl.ds(u * 16, 16)]
            pos = m >= jnp.uint32(0x80000000)
            uu = jnp.where(pos, m & jnp.uint32(0x7FFFFFFF), ~m)
            s = lax.bitcast_convert_type(uu, jnp.float32)
            gate_b[pl.ds(u * 16, 16)] = 1.0 / (1.0 + jnp.exp(-s))
            return carry

        lax.fori_loop(0, n_load // 16, gb, jnp.int32(0))
        return idx_sl

    def mult(rows_b, gate_b, n_write):
        def rb(i, carry):
            g = plsc.load_gather(gate_b, [jnp.full((16,), i, jnp.int32)])
            for u8 in range(_F // 16):
                rows_b[i, pl.ds(u8 * 16, 16)] = (
                    rows_b[i, pl.ds(u8 * 16, 16)] * g)
            return carry

        lax.fori_loop(0, n_write, rb, jnp.int32(0))

    def topk_block(b, p0, n_load, n_write):
        idx_sl = stage(idxb[0], dkst[0], gate[0], b, p0, n_load)
        rows_sl = rows[0] if n_load == 128 else rows[0].at[pl.ds(0, n_load)]
        pltpu.sync_copy(x_hbm.at[idx_sl], rows_sl)
        mult(rows[0], gate[0], n_write)
        out_sl = rows[0] if n_write == 128 else rows[0].at[pl.ds(0, n_write)]
        pltpu.sync_copy(out_sl, out_hbm.at[pl.ds(b * _K + p0, n_write)])

    for bl in range(2):
        b = c * 2 + bl
        radix_pass(b, _SHIFTS[0], True, None, None, sB_k, sB_i)
        radix_pass(b, _SHIFTS[1], False, sB_k, sB_i, sA_k, sA_i)
        radix_pass(b, _SHIFTS[2], False, sA_k, sA_i, sB_k, sB_i)
        radix_pass(b, _SHIFTS[3], False, sB_k, sB_i, sA_k, sA_i)

        gd = [None, None]
        wd = [None, None]
        stage(idxb[0], dkst[0], gate[0], b, t * 128, 128)
        gd[0] = pltpu.async_copy(x_hbm.at[idxb[0]], rows[0], gsem[0])
        for jj in range(1, 12):
            buf, pbuf = jj % 2, (jj - 1) % 2
            if jj >= 2:
                wd[buf].wait()
            stage(idxb[buf], dkst[buf], gate[buf], b, (t + jj * _T) * 128, 128)
            gd[buf] = pltpu.async_copy(x_hbm.at[idxb[buf]], rows[buf],
                                       gsem[buf])
            gd[pbuf].wait()
            mult(rows[pbuf], gate[pbuf], 128)
            wd[pbuf] = pltpu.async_copy(
                rows[pbuf],
                out_hbm.at[pl.ds(b * _K + (t + (jj - 1) * _T) * 128, 128)],
                wsem[pbuf])
        gd[1].wait()
        mult(rows[1], gate[1], 128)
        wd[1] = pltpu.async_copy(
            rows[1], out_hbm.at[pl.ds(b * _K + (t + 11 * _T) * 128, 128)],
            wsem[1])
        wd[0].wait()
        wd[1].wait()
        pl.when(t < _NBLK - 12 * _T)(
            functools.partial(topk_block, b, (t + 12 * _T) * 128, 128, 128))
        pl.when(t == 3)(
            functools.partial(topk_block, b, _NBLK * 128, 48, _TAIL))
        plsc.subcore_barrier()


_sc_sort_gather = pl.kernel(
    _sc_body,
    out_type=jax.ShapeDtypeStruct((_B * _K, _F), jnp.float32),
    mesh=plsc.VectorSubcoreMesh(core_axis_name="c", subcore_axis_name="s"),
    compiler_params=pltpu.CompilerParams(needs_layout_passes=False),
    scratch_types=[
        pltpu.VMEM_SHARED((_NPAD,), jnp.uint32),
        pltpu.VMEM_SHARED((_NPAD,), jnp.int32),
        pltpu.VMEM_SHARED((_NPAD,), jnp.uint32),
        pltpu.VMEM_SHARED((_NPAD,), jnp.int32),
        pltpu.VMEM_SHARED((_T * _R,), jnp.int32),
        pltpu.VMEM((_CH,), jnp.uint32),
        pltpu.VMEM((_CH,), jnp.int32),
        pltpu.VMEM((_CH,), jnp.int32),
        pltpu.VMEM((_CH,), jnp.int32),
        pltpu.VMEM((_R,), jnp.int32),
        pltpu.VMEM((_R,), jnp.int32),
        pltpu.VMEM((_R,), jnp.int32),
        pltpu.VMEM((_R,), jnp.int32),
        pltpu.VMEM((_T * _R,), jnp.int32),
        pltpu.VMEM((128,), jnp.int32),
        pltpu.VMEM((128,), jnp.int32),
        pltpu.VMEM((128,), jnp.uint32),
        pltpu.VMEM((128,), jnp.uint32),
        pltpu.VMEM((128,), jnp.float32),
        pltpu.VMEM((128,), jnp.float32),
        pltpu.VMEM((128, _F), jnp.float32),
        pltpu.VMEM((128, _F), jnp.float32),
        pltpu.SemaphoreType.DMA,
        pltpu.SemaphoreType.DMA,
        pltpu.SemaphoreType.DMA,
        pltpu.SemaphoreType.DMA,
    ],
)


def kernel(inputs, score_proj):
    dk = _scores_tc(inputs, score_proj).reshape(_B * _NPAD)
    xflat = inputs.reshape(_B * _N, _F)
    out = _sc_sort_gather(dk, xflat)
    return out.reshape(_B, _K, _F)
